# Initial kernel scaffold; baseline (speedup 1.0000x reference)
import jax
import jax.numpy as jnp
from jax.experimental import pallas as pl


def kernel(X, edge_index, edge_weight, W, b):
    raise NotImplementedError("write your pallas kernel here")


# speedup vs baseline: 3.0920x; 3.0920x over previous
import functools

import jax
import jax.numpy as jnp
from jax import lax
from jax.experimental import pallas as pl
from jax.experimental.pallas import tpu as pltpu
from jax.experimental.pallas import tpu_sc as plsc

N = 10000
E = 320000
D = 128
DH = D // 2
NC = 2
NS = 16
EPT = E // NS
CH = 80
NCHUNK = EPT // CH
RPT = 624
RTAIL = N - NS * RPT

_mesh = plsc.VectorSubcoreMesh(core_axis_name="c", subcore_axis_name="s")


@functools.partial(
    pl.kernel,
    mesh=_mesh,
    compiler_params=pltpu.CompilerParams(use_tc_tiling_on_sc=False),
    out_type=jax.ShapeDtypeStruct((NC, N, DH), jnp.float32),
    scratch_types=[
        pltpu.VMEM((NCHUNK, CH), jnp.int32),
        pltpu.VMEM((NCHUNK, CH), jnp.int32),
        pltpu.VMEM((CH, DH), jnp.float32),
        pltpu.VMEM((NCHUNK, CH), jnp.float32),
        pltpu.VMEM_SHARED((N, DH), jnp.float32),
        pltpu.SemaphoreType.DMA,
    ],
)
def _aggregate(x2_hbm, src2_hbm, dst_hbm, ew_hbm, out_hbm,
               src_v, dst_v, rows_v, ew_v, acc, sem):
    cc = lax.axis_index("c")
    ss = lax.axis_index("s")

    pltpu.sync_copy(src2_hbm.at[cc, ss], src_v)
    pltpu.sync_copy(dst_hbm.at[ss], dst_v)
    pltpu.sync_copy(ew_hbm.at[ss], ew_v)

    def _zrow(i, _):
        for j in range(DH // 16):
            rows_v[i, pl.ds(j * 16, 16)] = jnp.zeros((16,), jnp.float32)
        return 0
    lax.fori_loop(0, CH, _zrow, 0)
    for k in range(7):
        pltpu.sync_copy(rows_v, acc.at[pl.ds(ss * RPT + k * CH, CH)])
    pltpu.sync_copy(rows_v.at[pl.ds(0, RPT - 7 * CH)],
                    acc.at[pl.ds(ss * RPT + 7 * CH, RPT - 7 * CH)])

    @pl.when(ss == NS - 1)
    def _zero_tail():
        pltpu.sync_copy(rows_v.at[pl.ds(0, RTAIL)],
                        acc.at[pl.ds(NS * RPT, RTAIL)])

    plsc.subcore_barrier()

    def _chunk(ci, _):
        pltpu.async_copy(x2_hbm.at[src_v.at[ci]], rows_v, sem).wait()

        def _scale(g, _):
            wvec = ew_v[ci, pl.ds(g * 16, 16)]
            for l in range(16):
                e = g * 16 + l
                w = wvec[l]
                for j in range(DH // 16):
                    sl = pl.ds(j * 16, 16)
                    rows_v[e, sl] = rows_v[e, sl] * w
            return 0
        lax.fori_loop(0, CH // 16, _scale, 0)

        pltpu.sync_copy(rows_v, acc.at[dst_v.at[ci]], add=True)
        return 0

    lax.fori_loop(0, NCHUNK, _chunk, 0)
    plsc.subcore_barrier()

    pltpu.sync_copy(acc.at[pl.ds(ss * RPT, RPT)],
                    out_hbm.at[cc, pl.ds(ss * RPT, RPT)])

    @pl.when(ss == NS - 1)
    def _write_tail():
        pltpu.sync_copy(acc.at[pl.ds(NS * RPT, RTAIL)],
                        out_hbm.at[cc, pl.ds(NS * RPT, RTAIL)])


_BM = 1000


def _mm_body(p_ref, w_ref, b_ref, o_ref):
    o_ref[...] = (
        jnp.dot(p_ref[0], w_ref[0], preferred_element_type=jnp.float32)
        + jnp.dot(p_ref[1], w_ref[1], preferred_element_type=jnp.float32)
        + b_ref[...]
    )


def _finish(partials, W2, b2):
    return pl.pallas_call(
        _mm_body,
        grid=(N // _BM,),
        in_specs=[
            pl.BlockSpec((NC, _BM, DH), lambda i: (0, i, 0)),
            pl.BlockSpec((NC, DH, D), lambda i: (0, 0, 0)),
            pl.BlockSpec((1, D), lambda i: (0, 0)),
        ],
        out_specs=pl.BlockSpec((_BM, D), lambda i: (i, 0)),
        out_shape=jax.ShapeDtypeStruct((N, D), jnp.float32),
    )(partials, W2, b2)


def kernel(X, edge_index, edge_weight, W, b):
    src = edge_index[0].astype(jnp.int32)
    dst = edge_index[1].astype(jnp.int32).reshape(NS, NCHUNK, CH)
    ew = edge_weight.reshape(NS, NCHUNK, CH)
    src2 = jnp.stack([src * 2, src * 2 + 1]).reshape(NC, NS, NCHUNK, CH)
    x2 = X.reshape(NC * N, DH)
    partials = _aggregate(x2, src2, dst, ew)
    w2 = jnp.stack([W[:DH], W[DH:]])
    return _finish(partials, w2, b.reshape(1, D))


# speedup vs baseline: 4.4368x; 1.4349x over previous
import functools

import jax
import jax.numpy as jnp
from jax import lax
from jax.experimental import pallas as pl
from jax.experimental.pallas import tpu as pltpu
from jax.experimental.pallas import tpu_sc as plsc

N = 10000
E = 320000
D = 128
DH = D // 2
NC = 2
NS = 16
EPT = E // NS
CH = 80
NCHUNK = EPT // CH
RPT = 624
RTAIL = N - NS * RPT

_mesh = plsc.VectorSubcoreMesh(core_axis_name="c", subcore_axis_name="s")


@functools.partial(
    pl.kernel,
    mesh=_mesh,
    compiler_params=pltpu.CompilerParams(use_tc_tiling_on_sc=False),
    out_type=jax.ShapeDtypeStruct((NC, N, DH), jnp.float32),
    scratch_types=[
        pltpu.VMEM((NCHUNK, CH), jnp.int32),
        pltpu.VMEM((NCHUNK, CH), jnp.int32),
        pltpu.VMEM((CH, DH), jnp.float32),
        pltpu.VMEM((CH, DH), jnp.float32),
        pltpu.VMEM((NCHUNK, CH), jnp.float32),
        pltpu.VMEM_SHARED((N, DH), jnp.float32),
        pltpu.SemaphoreType.DMA,
        pltpu.SemaphoreType.DMA,
    ],
)
def _aggregate(x2_hbm, src2_hbm, dst_hbm, ew_hbm, out_hbm,
               src_v, dst_v, rows0_v, rows1_v, ew_v, acc, sem0, sem1):
    cc = lax.axis_index("c")
    ss = lax.axis_index("s")

    pltpu.sync_copy(src2_hbm.at[cc, ss], src_v)
    pltpu.sync_copy(dst_hbm.at[ss], dst_v)
    pltpu.sync_copy(ew_hbm.at[ss], ew_v)

    def _zrow(i, _):
        for j in range(DH // 16):
            rows0_v[i, pl.ds(j * 16, 16)] = jnp.zeros((16,), jnp.float32)
        return 0
    lax.fori_loop(0, CH, _zrow, 0)
    for k in range(7):
        pltpu.sync_copy(rows0_v, acc.at[pl.ds(ss * RPT + k * CH, CH)])
    pltpu.sync_copy(rows0_v.at[pl.ds(0, RPT - 7 * CH)],
                    acc.at[pl.ds(ss * RPT + 7 * CH, RPT - 7 * CH)])

    @pl.when(ss == NS - 1)
    def _zero_tail():
        pltpu.sync_copy(rows0_v.at[pl.ds(0, RTAIL)],
                        acc.at[pl.ds(NS * RPT, RTAIL)])

    plsc.subcore_barrier()

    def _scale_scatter(buf, ci):
        def _scale(g, _):
            wvec = ew_v[ci, pl.ds(g * 16, 16)]
            for l in range(16):
                e = g * 16 + l
                w = wvec[l]
                for j in range(DH // 16):
                    sl = pl.ds(j * 16, 16)
                    buf[e, sl] = buf[e, sl] * w
            return 0
        lax.fori_loop(0, CH // 16, _scale, 0)
        pltpu.sync_copy(buf, acc.at[dst_v.at[ci]], add=True)

    pltpu.async_copy(x2_hbm.at[src_v.at[0]], rows0_v, sem0)

    def _pair(i, _):
        c0 = i * 2
        pltpu.make_async_copy(x2_hbm.at[src_v.at[c0]], rows0_v, sem0).wait()
        pltpu.async_copy(x2_hbm.at[src_v.at[c0 + 1]], rows1_v, sem1)
        _scale_scatter(rows0_v, c0)
        pltpu.make_async_copy(x2_hbm.at[src_v.at[c0 + 1]], rows1_v, sem1).wait()

        @pl.when(i < NCHUNK // 2 - 1)
        def _next():
            pltpu.async_copy(x2_hbm.at[src_v.at[c0 + 2]], rows0_v, sem0)

        _scale_scatter(rows1_v, c0 + 1)
        return 0

    lax.fori_loop(0, NCHUNK // 2, _pair, 0)
    plsc.subcore_barrier()

    pltpu.sync_copy(acc.at[pl.ds(ss * RPT, RPT)],
                    out_hbm.at[cc, pl.ds(ss * RPT, RPT)])

    @pl.when(ss == NS - 1)
    def _write_tail():
        pltpu.sync_copy(acc.at[pl.ds(NS * RPT, RTAIL)],
                        out_hbm.at[cc, pl.ds(NS * RPT, RTAIL)])


_BM = 1000


def _mm_body(p_ref, w_ref, b_ref, o_ref):
    o_ref[...] = (
        jnp.dot(p_ref[0], w_ref[0], preferred_element_type=jnp.float32)
        + jnp.dot(p_ref[1], w_ref[1], preferred_element_type=jnp.float32)
        + b_ref[...]
    )


def _finish(partials, W2, b2):
    return pl.pallas_call(
        _mm_body,
        grid=(N // _BM,),
        in_specs=[
            pl.BlockSpec((NC, _BM, DH), lambda i: (0, i, 0)),
            pl.BlockSpec((NC, DH, D), lambda i: (0, 0, 0)),
            pl.BlockSpec((1, D), lambda i: (0, 0)),
        ],
        out_specs=pl.BlockSpec((_BM, D), lambda i: (i, 0)),
        out_shape=jax.ShapeDtypeStruct((N, D), jnp.float32),
    )(partials, W2, b2)


def kernel(X, edge_index, edge_weight, W, b):
    src = edge_index[0].astype(jnp.int32)
    dst = edge_index[1].astype(jnp.int32).reshape(NS, NCHUNK, CH)
    ew = edge_weight.reshape(NS, NCHUNK, CH)
    src2 = jnp.stack([src * 2, src * 2 + 1]).reshape(NC, NS, NCHUNK, CH)
    x2 = X.reshape(NC * N, DH)
    partials = _aggregate(x2, src2, dst, ew)
    w2 = jnp.stack([W[:DH], W[DH:]])
    return _finish(partials, w2, b.reshape(1, D))


# speedup vs baseline: 6.7247x; 1.5157x over previous
import functools

import jax
import jax.numpy as jnp
from jax import lax
from jax.experimental import pallas as pl
from jax.experimental.pallas import tpu as pltpu
from jax.experimental.pallas import tpu_sc as plsc

N = 10000
E = 320000
D = 128
DH = D // 2
NC = 2
NS = 16
EPT = E // NS
CH = 80
NCHUNK = EPT // CH
RPT = 624
RTAIL = N - NS * RPT

_mesh = plsc.VectorSubcoreMesh(core_axis_name="c", subcore_axis_name="s")


@functools.partial(
    pl.kernel,
    mesh=_mesh,
    compiler_params=pltpu.CompilerParams(use_tc_tiling_on_sc=False),
    out_type=jax.ShapeDtypeStruct((NC, N, DH), jnp.float32),
    scratch_types=[
        pltpu.VMEM((NCHUNK + 2, CH), jnp.int32),
        pltpu.VMEM((NCHUNK, CH), jnp.int32),
        pltpu.VMEM((CH, DH), jnp.float32),
        pltpu.VMEM((CH, DH), jnp.float32),
        pltpu.VMEM((CH, DH), jnp.float32),
        pltpu.VMEM((CH, DH), jnp.float32),
        pltpu.VMEM((NCHUNK, CH), jnp.float32),
        pltpu.VMEM_SHARED((N, DH), jnp.float32),
        pltpu.SemaphoreType.DMA,
        pltpu.SemaphoreType.DMA,
    ],
)
def _aggregate(x2_hbm, src2_hbm, dst_hbm, ew_hbm, out_hbm,
               src_v, dst_v, g0_v, g1_v, s0_v, s1_v, ew_v, acc, sem0, sem1):
    cc = lax.axis_index("c")
    ss = lax.axis_index("s")

    pltpu.sync_copy(src2_hbm.at[cc, ss], src_v.at[pl.ds(0, NCHUNK)])
    pltpu.sync_copy(dst_hbm.at[ss], dst_v)
    pltpu.sync_copy(ew_hbm.at[ss], ew_v)

    def _zpad(i, _):
        for j in range(CH // 16):
            src_v[NCHUNK + i, pl.ds(j * 16, 16)] = jnp.zeros((16,), jnp.int32)
        return 0
    lax.fori_loop(0, 2, _zpad, 0)

    def _zrow(i, _):
        for j in range(DH // 16):
            s0_v[i, pl.ds(j * 16, 16)] = jnp.zeros((16,), jnp.float32)
        return 0
    lax.fori_loop(0, CH, _zrow, 0)
    for k in range(7):
        pltpu.sync_copy(s0_v, acc.at[pl.ds(ss * RPT + k * CH, CH)])
    pltpu.sync_copy(s0_v.at[pl.ds(0, RPT - 7 * CH)],
                    acc.at[pl.ds(ss * RPT + 7 * CH, RPT - 7 * CH)])

    @pl.when(ss == NS - 1)
    def _zero_tail():
        pltpu.sync_copy(s0_v.at[pl.ds(0, RTAIL)],
                        acc.at[pl.ds(NS * RPT, RTAIL)])

    plsc.subcore_barrier()

    def _scale(gbuf, sbuf, ci):
        def _grp(g, _):
            wvec = ew_v[ci, pl.ds(g * 16, 16)]
            for l in range(16):
                e = g * 16 + l
                w = wvec[l]
                for j in range(DH // 16):
                    sl = pl.ds(j * 16, 16)
                    sbuf[e, sl] = gbuf[e, sl] * w
            return 0
        lax.fori_loop(0, CH // 16, _grp, 0)

    pltpu.async_copy(x2_hbm.at[src_v.at[0]], g0_v, sem0)
    pltpu.async_copy(x2_hbm.at[src_v.at[1]], g1_v, sem1)

    def _pair(i, _):
        c0 = i * 2
        pltpu.make_async_copy(x2_hbm.at[src_v.at[c0]], g0_v, sem0).wait()
        _scale(g0_v, s0_v, c0)
        pltpu.async_copy(x2_hbm.at[src_v.at[c0 + 2]], g0_v, sem0)
        pltpu.sync_copy(s0_v, acc.at[dst_v.at[c0]], add=True)

        pltpu.make_async_copy(x2_hbm.at[src_v.at[c0 + 1]], g1_v, sem1).wait()
        _scale(g1_v, s1_v, c0 + 1)
        pltpu.async_copy(x2_hbm.at[src_v.at[c0 + 3]], g1_v, sem1)
        pltpu.sync_copy(s1_v, acc.at[dst_v.at[c0 + 1]], add=True)
        return 0

    lax.fori_loop(0, NCHUNK // 2, _pair, 0)
    pltpu.make_async_copy(x2_hbm.at[src_v.at[NCHUNK]], g0_v, sem0).wait()
    pltpu.make_async_copy(x2_hbm.at[src_v.at[NCHUNK + 1]], g1_v, sem1).wait()
    plsc.subcore_barrier()

    pltpu.sync_copy(acc.at[pl.ds(ss * RPT, RPT)],
                    out_hbm.at[cc, pl.ds(ss * RPT, RPT)])

    @pl.when(ss == NS - 1)
    def _write_tail():
        pltpu.sync_copy(acc.at[pl.ds(NS * RPT, RTAIL)],
                        out_hbm.at[cc, pl.ds(NS * RPT, RTAIL)])


_BM = 1000


def _mm_body(p_ref, w_ref, b_ref, o_ref):
    o_ref[...] = (
        jnp.dot(p_ref[0], w_ref[0], preferred_element_type=jnp.float32)
        + jnp.dot(p_ref[1], w_ref[1], preferred_element_type=jnp.float32)
        + b_ref[...]
    )


def _finish(partials, W2, b2):
    return pl.pallas_call(
        _mm_body,
        grid=(N // _BM,),
        in_specs=[
            pl.BlockSpec((NC, _BM, DH), lambda i: (0, i, 0)),
            pl.BlockSpec((NC, DH, D), lambda i: (0, 0, 0)),
            pl.BlockSpec((1, D), lambda i: (0, 0)),
        ],
        out_specs=pl.BlockSpec((_BM, D), lambda i: (i, 0)),
        out_shape=jax.ShapeDtypeStruct((N, D), jnp.float32),
    )(partials, W2, b2)


def kernel(X, edge_index, edge_weight, W, b):
    src = edge_index[0].astype(jnp.int32)
    dst = edge_index[1].astype(jnp.int32).reshape(NS, NCHUNK, CH)
    ew = edge_weight.reshape(NS, NCHUNK, CH)
    src2 = jnp.stack([src * 2, src * 2 + 1]).reshape(NC, NS, NCHUNK, CH)
    x2 = X.reshape(NC * N, DH)
    partials = _aggregate(x2, src2, dst, ew)
    w2 = jnp.stack([W[:DH], W[DH:]])
    return _finish(partials, w2, b.reshape(1, D))
